# Initial kernel scaffold; baseline (speedup 1.0000x reference)
import jax
import jax.numpy as jnp
from jax.experimental import pallas as pl


def kernel(feat, edge_index, W_self, W_neigh, bias):
    raise NotImplementedError("write your pallas kernel here")


# speedup vs baseline: 3.9117x; 3.9117x over previous
import functools

import jax
import jax.numpy as jnp
from jax import lax
from jax.experimental import pallas as pl
from jax.experimental.pallas import tpu as pltpu
from jax.experimental.pallas import tpu_sc as plsc

N_NODES = 10000
D = 128
DH = D // 2
NC = 2
NS = 16
ROWS_PER_TILE = 640
N_PAD = NS * ROWS_PER_TILE
CHUNK = 80
ZROWS = 80


def _sc_aggregate(feat_lo, feat_hi, src, dst):
    E = src.shape[0]
    ept = E // NS
    n_chunks = ept // CHUNK

    mesh = plsc.VectorSubcoreMesh(core_axis_name="c", subcore_axis_name="s")

    @functools.partial(
        pl.kernel,
        mesh=mesh,
        compiler_params=pltpu.CompilerParams(use_tc_tiling_on_sc=False),
        out_type=[
            jax.ShapeDtypeStruct((NC, N_PAD, DH), jnp.float32),
            jax.ShapeDtypeStruct((N_PAD, 16), jnp.float32),
        ],
        scratch_types=[
            pltpu.VMEM((CHUNK,), jnp.int32),
            pltpu.VMEM((CHUNK,), jnp.int32),
            pltpu.VMEM((CHUNK, DH), jnp.float32),
            pltpu.VMEM((CHUNK, 16), jnp.float32),
            pltpu.VMEM((ZROWS, DH), jnp.float32),
            pltpu.VMEM((ROWS_PER_TILE, 16), jnp.float32),
            pltpu.VMEM_SHARED((N_PAD, DH), jnp.float32),
            pltpu.VMEM_SHARED((N_PAD, 16), jnp.float32),
            pltpu.SemaphoreType.DMA,
        ],
    )
    def agg(lo_hbm, hi_hbm, src_hbm, dst_hbm, sums_hbm, deg_hbm,
            src_v, dst_v, rows_v, ones_v, zbuf_v, zdeg_v, acc_sh, deg_sh, sem):
        cid = lax.axis_index("c")
        sid = lax.axis_index("s")

        zeros16 = jnp.zeros((16,), jnp.float32)
        ones16 = jnp.ones((16,), jnp.float32)

        def zero_zbuf(i, _):
            for j in range(DH // 16):
                zbuf_v[i, pl.ds(j * 16, 16)] = zeros16
            return _
        lax.fori_loop(0, ZROWS, zero_zbuf, None)

        def zero_zdeg(i, _):
            zdeg_v[i, pl.ds(0, 16)] = zeros16
            return _
        lax.fori_loop(0, ROWS_PER_TILE, zero_zdeg, None)

        def fill_ones(i, _):
            ones_v[i, pl.ds(0, 16)] = ones16
            return _
        lax.fori_loop(0, CHUNK, fill_ones, None)

        for b in range(ROWS_PER_TILE // ZROWS):
            pltpu.sync_copy(zbuf_v,
                            acc_sh.at[pl.ds(sid * ROWS_PER_TILE + b * ZROWS, ZROWS)])
        pltpu.sync_copy(zdeg_v, deg_sh.at[pl.ds(sid * ROWS_PER_TILE, ROWS_PER_TILE)])
        plsc.subcore_barrier()

        def edge_chunk(c, _):
            base = sid * ept + c * CHUNK
            pltpu.sync_copy(src_hbm.at[pl.ds(base, CHUNK)], src_v)
            pltpu.sync_copy(dst_hbm.at[pl.ds(base, CHUNK)], dst_v)
            @pl.when(cid == 0)
            def _():
                pltpu.async_copy(lo_hbm.at[src_v], rows_v, sem).wait()
            @pl.when(cid != 0)
            def _():
                pltpu.async_copy(hi_hbm.at[src_v], rows_v, sem).wait()
            pltpu.sync_copy(rows_v, acc_sh.at[dst_v], add=True)
            @pl.when(cid == 0)
            def _():
                pltpu.sync_copy(ones_v, deg_sh.at[dst_v], add=True)
            return _
        lax.fori_loop(0, n_chunks, edge_chunk, None)

        plsc.subcore_barrier()
        pltpu.sync_copy(acc_sh.at[pl.ds(sid * ROWS_PER_TILE, ROWS_PER_TILE)],
                        sums_hbm.at[cid, pl.ds(sid * ROWS_PER_TILE, ROWS_PER_TILE)])
        @pl.when(cid == 0)
        def _():
            pltpu.sync_copy(deg_sh.at[pl.ds(sid * ROWS_PER_TILE, ROWS_PER_TILE)],
                            deg_hbm.at[pl.ds(sid * ROWS_PER_TILE, ROWS_PER_TILE)])

    return agg(feat_lo, feat_hi, src, dst)


def _combine(feat, sums, deg, W_self, W_neigh, bias2d):
    R = 1000
    dn = (((1,), (1,)), ((), ()))

    def body(feat_ref, sums_ref, deg_ref, ws_ref, wn_ref, b_ref, o_ref):
        d = deg_ref[:, :1]
        s = jnp.concatenate([sums_ref[0], sums_ref[1]], axis=1)
        hn = s * (1.0 / jnp.maximum(d, 1.0))
        o_ref[...] = (
            lax.dot_general(feat_ref[...], ws_ref[...], dn,
                            preferred_element_type=jnp.float32)
            + lax.dot_general(hn, wn_ref[...], dn,
                              preferred_element_type=jnp.float32)
            + b_ref[...]
        )

    return pl.pallas_call(
        body,
        grid=(N_NODES // R,),
        in_specs=[
            pl.BlockSpec((R, D), lambda i: (i, 0)),
            pl.BlockSpec((NC, R, DH), lambda i: (0, i, 0)),
            pl.BlockSpec((R, 16), lambda i: (i, 0)),
            pl.BlockSpec((D, D), lambda i: (0, 0)),
            pl.BlockSpec((D, D), lambda i: (0, 0)),
            pl.BlockSpec((1, D), lambda i: (0, 0)),
        ],
        out_specs=pl.BlockSpec((R, D), lambda i: (i, 0)),
        out_shape=jax.ShapeDtypeStruct((N_NODES, D), jnp.float32),
    )(feat, sums, deg, W_self, W_neigh, bias2d)


def kernel(feat, edge_index, W_self, W_neigh, bias):
    feat_lo = feat[:, :DH]
    feat_hi = feat[:, DH:]
    sums, deg = _sc_aggregate(feat_lo, feat_hi, edge_index[0], edge_index[1])
    return _combine(feat, sums, deg, W_self, W_neigh, bias.reshape(1, D))


# speedup vs baseline: 6.1689x; 1.5770x over previous
import functools

import jax
import jax.numpy as jnp
from jax import lax
from jax.experimental import pallas as pl
from jax.experimental.pallas import tpu as pltpu
from jax.experimental.pallas import tpu_sc as plsc

N_NODES = 10000
D = 128
DH = D // 2
NC = 2
NS = 16
ROWS_PER_TILE = 640
N_PAD = NS * ROWS_PER_TILE
CHUNK = 80
ZROWS = 80


def _sc_aggregate(fhalf, src2d, dst2d):
    n_rows = src2d.shape[0]
    cpt = n_rows // NS
    half = cpt // 2

    mesh = plsc.VectorSubcoreMesh(core_axis_name="c", subcore_axis_name="s")

    @functools.partial(
        pl.kernel,
        mesh=mesh,
        compiler_params=pltpu.CompilerParams(use_tc_tiling_on_sc=False),
        out_type=[
            jax.ShapeDtypeStruct((NC, N_PAD, DH), jnp.float32),
            jax.ShapeDtypeStruct((NC, N_PAD, 16), jnp.float32),
        ],
        scratch_types=[
            pltpu.VMEM((cpt, CHUNK), jnp.int32),
            pltpu.VMEM((cpt, CHUNK), jnp.int32),
            pltpu.VMEM((CHUNK,), jnp.int32),
            pltpu.VMEM((CHUNK,), jnp.int32),
            pltpu.VMEM((CHUNK, DH), jnp.float32),
            pltpu.VMEM((CHUNK, DH), jnp.float32),
            pltpu.VMEM((CHUNK, 16), jnp.float32),
            pltpu.VMEM((ZROWS, DH), jnp.float32),
            pltpu.VMEM((ROWS_PER_TILE, 16), jnp.float32),
            pltpu.VMEM_SHARED((N_PAD, DH), jnp.float32),
            pltpu.VMEM_SHARED((N_PAD, 16), jnp.float32),
            pltpu.SemaphoreType.DMA,
            pltpu.SemaphoreType.DMA,
            pltpu.SemaphoreType.DMA,
            pltpu.SemaphoreType.DMA,
        ],
    )
    def agg(lo_hbm, hi_hbm, src_hbm, dst_hbm, sums_hbm, deg_hbm,
            src_all, dst_all, src_v, dst_v, rows0, rows1, ones_v, zbuf_v, zdeg_v,
            acc_sh, deg_sh, sem_i, sem_g0, sem_g1, sem_deg):
        cid = lax.axis_index("c")
        sid = lax.axis_index("s")
        rows = (rows0, rows1)
        sem_g = (sem_g0, sem_g1)

        pltpu.async_copy(src_hbm.at[pl.ds(sid * cpt, cpt)], src_all, sem_i)
        pltpu.async_copy(dst_hbm.at[pl.ds(sid * cpt, cpt)], dst_all, sem_i)

        zeros16 = jnp.zeros((16,), jnp.float32)
        ones16 = jnp.ones((16,), jnp.float32)

        def zero_zbuf(i, _):
            for j in range(DH // 16):
                zbuf_v[i, pl.ds(j * 16, 16)] = zeros16
            return _
        lax.fori_loop(0, ZROWS, zero_zbuf, None)

        def zero_zdeg(i, _):
            zdeg_v[i, pl.ds(0, 16)] = zeros16
            return _
        lax.fori_loop(0, ROWS_PER_TILE, zero_zdeg, None)

        def fill_ones(i, _):
            ones_v[i, pl.ds(0, 16)] = ones16
            return _
        lax.fori_loop(0, CHUNK, fill_ones, None)

        for b in range(ROWS_PER_TILE // ZROWS):
            pltpu.sync_copy(zbuf_v,
                            acc_sh.at[pl.ds(sid * ROWS_PER_TILE + b * ZROWS, ZROWS)])
        pltpu.sync_copy(zdeg_v, deg_sh.at[pl.ds(sid * ROWS_PER_TILE, ROWS_PER_TILE)])

        pltpu.make_async_copy(src_hbm.at[pl.ds(sid * cpt, cpt)], src_all, sem_i).wait()
        pltpu.make_async_copy(dst_hbm.at[pl.ds(sid * cpt, cpt)], dst_all, sem_i).wait()
        plsc.subcore_barrier()

        def chunk_body(c, _):
            for g in range(CHUNK // 16):
                src_v[pl.ds(g * 16, 16)] = src_all[c, pl.ds(g * 16, 16)]
                dst_v[pl.ds(g * 16, 16)] = dst_all[c, pl.ds(g * 16, 16)]
            @pl.when(cid == 0)
            def _():
                pltpu.async_copy(lo_hbm.at[src_v], rows0, sem_g0).wait()
            @pl.when(cid != 0)
            def _():
                pltpu.async_copy(hi_hbm.at[src_v], rows0, sem_g0).wait()
            pltpu.sync_copy(rows0, acc_sh.at[dst_v], add=True)

            @pl.when((c < half) == (cid == 0))
            def _():
                pltpu.sync_copy(ones_v, deg_sh.at[dst_v], add=True)
            return _
        lax.fori_loop(0, cpt, chunk_body, None)

        plsc.subcore_barrier()
        pltpu.sync_copy(acc_sh.at[pl.ds(sid * ROWS_PER_TILE, ROWS_PER_TILE)],
                        sums_hbm.at[cid, pl.ds(sid * ROWS_PER_TILE, ROWS_PER_TILE)])
        pltpu.sync_copy(deg_sh.at[pl.ds(sid * ROWS_PER_TILE, ROWS_PER_TILE)],
                        deg_hbm.at[cid, pl.ds(sid * ROWS_PER_TILE, ROWS_PER_TILE)])

    return agg(fhalf[0], fhalf[1], src2d, dst2d)


def _combine(feat, sums, deg, W_self, W_neigh, bias2d):
    R = 1000
    dn = (((1,), (1,)), ((), ()))

    def body(feat_ref, sums_ref, deg_ref, ws_ref, wn_ref, b_ref, o_ref):
        d = deg_ref[0][:, :1] + deg_ref[1][:, :1]
        s = jnp.concatenate([sums_ref[0], sums_ref[1]], axis=1)
        hn = s * (1.0 / jnp.maximum(d, 1.0))
        o_ref[...] = (
            lax.dot_general(feat_ref[...], ws_ref[...], dn,
                            preferred_element_type=jnp.float32)
            + lax.dot_general(hn, wn_ref[...], dn,
                              preferred_element_type=jnp.float32)
            + b_ref[...]
        )

    return pl.pallas_call(
        body,
        grid=(N_NODES // R,),
        in_specs=[
            pl.BlockSpec((R, D), lambda i: (i, 0)),
            pl.BlockSpec((NC, R, DH), lambda i: (0, i, 0)),
            pl.BlockSpec((NC, R, 16), lambda i: (0, i, 0)),
            pl.BlockSpec((D, D), lambda i: (0, 0)),
            pl.BlockSpec((D, D), lambda i: (0, 0)),
            pl.BlockSpec((1, D), lambda i: (0, 0)),
        ],
        out_specs=pl.BlockSpec((R, D), lambda i: (i, 0)),
        out_shape=jax.ShapeDtypeStruct((N_NODES, D), jnp.float32),
    )(feat, sums, deg, W_self, W_neigh, bias2d)


def kernel(feat, edge_index, W_self, W_neigh, bias):
    E = edge_index.shape[1]
    fhalf = jnp.stack([feat[:, :DH], feat[:, DH:]])
    src2d = edge_index[0].reshape(E // CHUNK, CHUNK)
    dst2d = edge_index[1].reshape(E // CHUNK, CHUNK)
    sums, deg = _sc_aggregate(fhalf, src2d, dst2d)
    return _combine(feat, sums, deg, W_self, W_neigh, bias.reshape(1, D))


# speedup vs baseline: 9.8994x; 1.6047x over previous
import functools

import jax
import jax.numpy as jnp
from jax import lax
from jax.experimental import pallas as pl
from jax.experimental.pallas import tpu as pltpu
from jax.experimental.pallas import tpu_sc as plsc

N_NODES = 10000
D = 128
DH = D // 2
NC = 2
NS = 16
ROWS_PER_TILE = 640
N_PAD = NS * ROWS_PER_TILE
CHUNK = 80
ZROWS = 80


def _sc_aggregate(fhalf, src2d, dst2d):
    n_rows = src2d.shape[0]
    cpt = n_rows // NS
    half = cpt // 2

    mesh = plsc.VectorSubcoreMesh(core_axis_name="c", subcore_axis_name="s")

    @functools.partial(
        pl.kernel,
        mesh=mesh,
        compiler_params=pltpu.CompilerParams(use_tc_tiling_on_sc=False),
        out_type=[
            jax.ShapeDtypeStruct((NC, N_PAD, DH), jnp.float32),
            jax.ShapeDtypeStruct((NC, N_PAD, 16), jnp.float32),
        ],
        scratch_types=[
            pltpu.VMEM((cpt, CHUNK), jnp.int32),
            pltpu.VMEM((cpt, CHUNK), jnp.int32),
            pltpu.VMEM((CHUNK,), jnp.int32),
            pltpu.VMEM((CHUNK,), jnp.int32),
            pltpu.VMEM((CHUNK,), jnp.int32),
            pltpu.VMEM((CHUNK,), jnp.int32),
            pltpu.VMEM((CHUNK, DH), jnp.float32),
            pltpu.VMEM((CHUNK, DH), jnp.float32),
            pltpu.VMEM((CHUNK, 16), jnp.float32),
            pltpu.VMEM((ZROWS, DH), jnp.float32),
            pltpu.VMEM((ROWS_PER_TILE, 16), jnp.float32),
            pltpu.VMEM_SHARED((N_PAD, DH), jnp.float32),
            pltpu.VMEM_SHARED((N_PAD, 16), jnp.float32),
            pltpu.SemaphoreType.DMA,
            pltpu.SemaphoreType.DMA,
            pltpu.SemaphoreType.DMA,
            pltpu.SemaphoreType.DMA,
        ],
    )
    def agg(lo_hbm, hi_hbm, src_hbm, dst_hbm, sums_hbm, deg_hbm,
            src_all, dst_all, src_v0, dst_v0, src_v1, dst_v1,
            rows0, rows1, ones_v, zbuf_v, zdeg_v,
            acc_sh, deg_sh, sem_i, sem_g0, sem_g1, sem_deg):
        cid = lax.axis_index("c")
        sid = lax.axis_index("s")
        rows = (rows0, rows1)
        sem_g = (sem_g0, sem_g1)
        src_v = (src_v0, src_v1)
        dst_v = (dst_v0, dst_v1)

        pltpu.async_copy(src_hbm.at[pl.ds(sid * cpt, cpt)], src_all, sem_i)
        pltpu.async_copy(dst_hbm.at[pl.ds(sid * cpt, cpt)], dst_all, sem_i)

        zeros16 = jnp.zeros((16,), jnp.float32)
        ones16 = jnp.ones((16,), jnp.float32)

        def zero_zbuf(i, _):
            for j in range(DH // 16):
                zbuf_v[i, pl.ds(j * 16, 16)] = zeros16
            return _
        lax.fori_loop(0, ZROWS, zero_zbuf, None)

        def zero_zdeg(i, _):
            zdeg_v[i, pl.ds(0, 16)] = zeros16
            return _
        lax.fori_loop(0, ROWS_PER_TILE, zero_zdeg, None)

        def fill_ones(i, _):
            ones_v[i, pl.ds(0, 16)] = ones16
            return _
        lax.fori_loop(0, CHUNK, fill_ones, None)

        for b in range(ROWS_PER_TILE // ZROWS):
            pltpu.sync_copy(zbuf_v,
                            acc_sh.at[pl.ds(sid * ROWS_PER_TILE + b * ZROWS, ZROWS)])
        pltpu.sync_copy(zdeg_v, deg_sh.at[pl.ds(sid * ROWS_PER_TILE, ROWS_PER_TILE)])

        pltpu.make_async_copy(src_hbm.at[pl.ds(sid * cpt, cpt)], src_all, sem_i).wait()
        pltpu.make_async_copy(dst_hbm.at[pl.ds(sid * cpt, cpt)], dst_all, sem_i).wait()
        plsc.subcore_barrier()

        def stage(c, b):
            for g in range(CHUNK // 16):
                src_v[b][pl.ds(g * 16, 16)] = src_all[c, pl.ds(g * 16, 16)]
                dst_v[b][pl.ds(g * 16, 16)] = dst_all[c, pl.ds(g * 16, 16)]

        def issue_gather(b):
            @pl.when(cid == 0)
            def _():
                pltpu.async_copy(lo_hbm.at[src_v[b]], rows[b], sem_g[b])
            @pl.when(cid != 0)
            def _():
                pltpu.async_copy(hi_hbm.at[src_v[b]], rows[b], sem_g[b])

        for b in range(2):
            stage(b, b)
            issue_gather(b)

        def pair(g, _):
            for b in range(2):
                c = g * 2 + b
                pltpu.make_async_copy(lo_hbm.at[src_v[b]], rows[b], sem_g[b]).wait()
                pltpu.sync_copy(rows[b], acc_sh.at[dst_v[b]], add=True)

                @pl.when((c < half) == (cid == 0))
                def _():
                    pltpu.sync_copy(ones_v, deg_sh.at[dst_v[b]], add=True)

                @pl.when(c + 2 < cpt)
                def _():
                    stage(c + 2, b)
                    issue_gather(b)
            return _
        lax.fori_loop(0, cpt // 2, pair, None)

        plsc.subcore_barrier()
        pltpu.sync_copy(acc_sh.at[pl.ds(sid * ROWS_PER_TILE, ROWS_PER_TILE)],
                        sums_hbm.at[cid, pl.ds(sid * ROWS_PER_TILE, ROWS_PER_TILE)])
        pltpu.sync_copy(deg_sh.at[pl.ds(sid * ROWS_PER_TILE, ROWS_PER_TILE)],
                        deg_hbm.at[cid, pl.ds(sid * ROWS_PER_TILE, ROWS_PER_TILE)])

    return agg(fhalf[0], fhalf[1], src2d, dst2d)


def _combine(feat, sums, deg, W_self, W_neigh, bias2d):
    R = 1000
    dn = (((1,), (1,)), ((), ()))

    def body(feat_ref, sums_ref, deg_ref, ws_ref, wn_ref, b_ref, o_ref):
        d = deg_ref[0][:, :1] + deg_ref[1][:, :1]
        s = jnp.concatenate([sums_ref[0], sums_ref[1]], axis=1)
        hn = s * (1.0 / jnp.maximum(d, 1.0))
        o_ref[...] = (
            lax.dot_general(feat_ref[...], ws_ref[...], dn,
                            preferred_element_type=jnp.float32)
            + lax.dot_general(hn, wn_ref[...], dn,
                              preferred_element_type=jnp.float32)
            + b_ref[...]
        )

    return pl.pallas_call(
        body,
        grid=(N_NODES // R,),
        in_specs=[
            pl.BlockSpec((R, D), lambda i: (i, 0)),
            pl.BlockSpec((NC, R, DH), lambda i: (0, i, 0)),
            pl.BlockSpec((NC, R, 16), lambda i: (0, i, 0)),
            pl.BlockSpec((D, D), lambda i: (0, 0)),
            pl.BlockSpec((D, D), lambda i: (0, 0)),
            pl.BlockSpec((1, D), lambda i: (0, 0)),
        ],
        out_specs=pl.BlockSpec((R, D), lambda i: (i, 0)),
        out_shape=jax.ShapeDtypeStruct((N_NODES, D), jnp.float32),
    )(feat, sums, deg, W_self, W_neigh, bias2d)


def kernel(feat, edge_index, W_self, W_neigh, bias):
    E = edge_index.shape[1]
    fhalf = jnp.stack([feat[:, :DH], feat[:, DH:]])
    src2d = edge_index[0].reshape(E // CHUNK, CHUNK)
    dst2d = edge_index[1].reshape(E // CHUNK, CHUNK)
    sums, deg = _sc_aggregate(fhalf, src2d, dst2d)
    return _combine(feat, sums, deg, W_self, W_neigh, bias.reshape(1, D))


# speedup vs baseline: 10.4139x; 1.0520x over previous
import functools

import jax
import jax.numpy as jnp
from jax import lax
from jax.experimental import pallas as pl
from jax.experimental.pallas import tpu as pltpu
from jax.experimental.pallas import tpu_sc as plsc

N_NODES = 10000
D = 128
DH = D // 2
NC = 2
NS = 16
ROWS_PER_TILE = 640
N_PAD = NS * ROWS_PER_TILE
CHUNK = 80
ZROWS = 80


def _sc_aggregate(fhalf, src2d, dst2d):
    n_rows = src2d.shape[0]
    cpt = n_rows // NS
    half = cpt // 2

    mesh = plsc.VectorSubcoreMesh(core_axis_name="c", subcore_axis_name="s")

    @functools.partial(
        pl.kernel,
        mesh=mesh,
        compiler_params=pltpu.CompilerParams(use_tc_tiling_on_sc=False),
        out_type=[
            jax.ShapeDtypeStruct((NC, N_PAD, DH), jnp.float32),
            jax.ShapeDtypeStruct((NC, N_PAD, 16), jnp.float32),
        ],
        scratch_types=[
            pltpu.VMEM((cpt, CHUNK), jnp.int32),
            pltpu.VMEM((cpt, CHUNK), jnp.int32),
            pltpu.VMEM((CHUNK,), jnp.int32),
            pltpu.VMEM((CHUNK,), jnp.int32),
            pltpu.VMEM((CHUNK,), jnp.int32),
            pltpu.VMEM((CHUNK,), jnp.int32),
            pltpu.VMEM((CHUNK, DH), jnp.float32),
            pltpu.VMEM((CHUNK, DH), jnp.float32),
            pltpu.VMEM((CHUNK, 16), jnp.float32),
            pltpu.VMEM((ZROWS, DH), jnp.float32),
            pltpu.VMEM((ROWS_PER_TILE, 16), jnp.float32),
            pltpu.VMEM_SHARED((N_PAD, DH), jnp.float32),
            pltpu.VMEM_SHARED((N_PAD, 16), jnp.float32),
            pltpu.SemaphoreType.DMA,
            pltpu.SemaphoreType.DMA,
            pltpu.SemaphoreType.DMA,
            pltpu.SemaphoreType.DMA,
        ],
    )
    def agg(f2_hbm, src_hbm, dst_hbm, sums_hbm, deg_hbm,
            src_all, dst_all, src_v0, dst_v0, src_v1, dst_v1,
            rows0, rows1, ones_v, zbuf_v, zdeg_v,
            acc_sh, deg_sh, sem_i, sem_g0, sem_g1, sem_deg):
        cid = lax.axis_index("c")
        sid = lax.axis_index("s")
        rows = (rows0, rows1)
        sem_g = (sem_g0, sem_g1)
        src_v = (src_v0, src_v1)
        dst_v = (dst_v0, dst_v1)

        pltpu.async_copy(src_hbm.at[pl.ds(sid * cpt, cpt)], src_all, sem_i)
        pltpu.async_copy(dst_hbm.at[pl.ds(sid * cpt, cpt)], dst_all, sem_i)

        zeros16 = jnp.zeros((16,), jnp.float32)
        ones16 = jnp.ones((16,), jnp.float32)

        def zero_zbuf(i, _):
            for j in range(DH // 16):
                zbuf_v[i, pl.ds(j * 16, 16)] = zeros16
            return _
        lax.fori_loop(0, ZROWS, zero_zbuf, None)

        def zero_zdeg(i, _):
            zdeg_v[i, pl.ds(0, 16)] = zeros16
            return _
        lax.fori_loop(0, ROWS_PER_TILE, zero_zdeg, None)

        def fill_ones(i, _):
            ones_v[i, pl.ds(0, 16)] = ones16
            return _
        lax.fori_loop(0, CHUNK, fill_ones, None)

        for b in range(ROWS_PER_TILE // ZROWS):
            pltpu.sync_copy(zbuf_v,
                            acc_sh.at[pl.ds(sid * ROWS_PER_TILE + b * ZROWS, ZROWS)])
        pltpu.sync_copy(zdeg_v, deg_sh.at[pl.ds(sid * ROWS_PER_TILE, ROWS_PER_TILE)])

        pltpu.make_async_copy(src_hbm.at[pl.ds(sid * cpt, cpt)], src_all, sem_i).wait()
        pltpu.make_async_copy(dst_hbm.at[pl.ds(sid * cpt, cpt)], dst_all, sem_i).wait()
        plsc.subcore_barrier()

        def stage(c, b):
            for g in range(CHUNK // 16):
                src_v[b][pl.ds(g * 16, 16)] = (
                    src_all[c, pl.ds(g * 16, 16)] * 2 + cid)
                dst_v[b][pl.ds(g * 16, 16)] = dst_all[c, pl.ds(g * 16, 16)]

        def issue_gather(b):
            pltpu.async_copy(f2_hbm.at[src_v[b]], rows[b], sem_g[b])

        for b in range(2):
            stage(b, b)
            issue_gather(b)

        def pair(g, _):
            for b in range(2):
                c = g * 2 + b
                pltpu.make_async_copy(f2_hbm.at[src_v[b]], rows[b], sem_g[b]).wait()
                pltpu.sync_copy(rows[b], acc_sh.at[dst_v[b]], add=True)

                @pl.when((c < half) == (cid == 0))
                def _():
                    pltpu.sync_copy(ones_v, deg_sh.at[dst_v[b]], add=True)

                @pl.when(c + 2 < cpt)
                def _():
                    stage(c + 2, b)
                    issue_gather(b)
            return _
        lax.fori_loop(0, cpt // 2, pair, None)

        plsc.subcore_barrier()
        pltpu.sync_copy(acc_sh.at[pl.ds(sid * ROWS_PER_TILE, ROWS_PER_TILE)],
                        sums_hbm.at[cid, pl.ds(sid * ROWS_PER_TILE, ROWS_PER_TILE)])
        pltpu.sync_copy(deg_sh.at[pl.ds(sid * ROWS_PER_TILE, ROWS_PER_TILE)],
                        deg_hbm.at[cid, pl.ds(sid * ROWS_PER_TILE, ROWS_PER_TILE)])

    return agg(fhalf, src2d, dst2d)


def _combine(feat, sums, deg, W_self, W_neigh, bias2d):
    R = 1000
    dn = (((1,), (1,)), ((), ()))

    def body(feat_ref, sums_ref, deg_ref, ws_ref, wn_ref, b_ref, o_ref):
        d = deg_ref[0][:, :1] + deg_ref[1][:, :1]
        s = jnp.concatenate([sums_ref[0], sums_ref[1]], axis=1)
        hn = s * (1.0 / jnp.maximum(d, 1.0))
        o_ref[...] = (
            lax.dot_general(feat_ref[...], ws_ref[...], dn,
                            preferred_element_type=jnp.float32)
            + lax.dot_general(hn, wn_ref[...], dn,
                              preferred_element_type=jnp.float32)
            + b_ref[...]
        )

    return pl.pallas_call(
        body,
        grid=(N_NODES // R,),
        in_specs=[
            pl.BlockSpec((R, D), lambda i: (i, 0)),
            pl.BlockSpec((NC, R, DH), lambda i: (0, i, 0)),
            pl.BlockSpec((NC, R, 16), lambda i: (0, i, 0)),
            pl.BlockSpec((D, D), lambda i: (0, 0)),
            pl.BlockSpec((D, D), lambda i: (0, 0)),
            pl.BlockSpec((1, D), lambda i: (0, 0)),
        ],
        out_specs=pl.BlockSpec((R, D), lambda i: (i, 0)),
        out_shape=jax.ShapeDtypeStruct((N_NODES, D), jnp.float32),
    )(feat, sums, deg, W_self, W_neigh, bias2d)


def kernel(feat, edge_index, W_self, W_neigh, bias):
    E = edge_index.shape[1]
    fhalf = feat.reshape(2 * N_NODES, DH)
    src2d = edge_index[0].reshape(E // CHUNK, CHUNK)
    dst2d = edge_index[1].reshape(E // CHUNK, CHUNK)
    sums, deg = _sc_aggregate(fhalf, src2d, dst2d)
    return _combine(feat, sums, deg, W_self, W_neigh, bias.reshape(1, D))
